# baseline (speedup 1.0000x reference)
import functools

import jax
import jax.numpy as jnp
from jax import lax
from jax.experimental import pallas as pl
from jax.experimental.pallas import tpu as pltpu
from jax.experimental.pallas import tpu_sc as plsc

VOCAB = 1000000
EMBED = 64
B = 1024
L = 200
N = B * L

_INFO = plsc.get_sparse_core_info()
NC = _INFO.num_cores
NS = _INFO.num_subcores
NW = NC * NS
PER_W = N // NW
SUB = 128
NSUB = PER_W // SUB
VL = 16
NV = EMBED // VL


def _xlane_sum(v):
    dnums = lax.GatherDimensionNumbers(
        offset_dims=(), collapsed_slice_dims=(0,), start_index_map=(0,))
    for s in (1, 2, 4, 8):
        idx = lax.iota(jnp.int32, VL) ^ s
        v = v + lax.gather(
            v, idx[:, None], dnums, slice_sizes=(1,),
            mode=lax.GatherScatterMode.PROMISE_IN_BOUNDS)
    return v


def _rsqrt_nr(x):
    i = plsc.bitcast(x, jnp.int32)
    i = jnp.int32(0x5F3759DF) - lax.shift_right_arithmetic(i, jnp.int32(1))
    y = plsc.bitcast(i, jnp.float32)
    half = x * 0.5
    for _ in range(3):
        y = y * (1.5 - half * y * y)
    return y


def _ln_kernel(ids_hbm, table_hbm, gamma_hbm, beta_hbm, out_hbm,
               idx_v, rows_v, gb_v, sem):
    wid = lax.axis_index("s") * NC + lax.axis_index("c")

    pltpu.sync_copy(gamma_hbm, gb_v.at[0])
    pltpu.sync_copy(beta_hbm, gb_v.at[1])

    gamma = [gb_v[0, pl.ds(k * VL, VL)] for k in range(NV)]
    beta = [gb_v[1, pl.ds(k * VL, VL)] for k in range(NV)]

    def subchunk(j, _):
        pltpu.sync_copy(
            ids_hbm.at[pl.ds(wid * PER_W + j * SUB, SUB)], idx_v)
        pltpu.async_copy(table_hbm.at[idx_v], rows_v, sem).wait()

        def row(r, _):
            x = [rows_v[r, pl.ds(k * VL, VL)] for k in range(NV)]
            s = x[0] + x[1] + x[2] + x[3]
            mean = _xlane_sum(s) * (1.0 / EMBED)
            d = [xk - mean for xk in x]
            sq = d[0] * d[0] + d[1] * d[1] + d[2] * d[2] + d[3] * d[3]
            var = _xlane_sum(sq) * (1.0 / EMBED)
            inv = _rsqrt_nr(var + 1e-12)
            for k in range(NV):
                rows_v[r, pl.ds(k * VL, VL)] = d[k] * inv * gamma[k] + beta[k]
            return 0

        lax.fori_loop(0, SUB, row, 0)
        pltpu.sync_copy(
            rows_v, out_hbm.at[pl.ds(wid * PER_W + j * SUB, SUB)])
        return 0

    lax.fori_loop(0, NSUB, subchunk, 0)


@jax.jit
def _run(ids2d, table, gamma, beta):
    mesh = plsc.VectorSubcoreMesh(core_axis_name="c", subcore_axis_name="s")
    out = pl.kernel(
        _ln_kernel,
        mesh=mesh,
        out_type=jax.ShapeDtypeStruct((N, EMBED), jnp.float32),
        scratch_types=[
            pltpu.VMEM((SUB,), jnp.int32),
            pltpu.VMEM((SUB, EMBED), jnp.float32),
            pltpu.VMEM((2, EMBED), jnp.float32),
            pltpu.SemaphoreType.DMA,
        ],
        compiler_params=pltpu.CompilerParams(
            needs_layout_passes=False, use_tc_tiling_on_sc=False),
    )(ids2d, table, gamma, beta)
    return out.reshape(B, L, EMBED)


def kernel(input_ids, word_table, ln_gamma, ln_beta):
    ids_flat = input_ids.astype(jnp.int32).reshape(N)
    return _run(ids_flat, word_table, ln_gamma, ln_beta)


# speedup vs baseline: 1.3019x; 1.3019x over previous
import functools

import jax
import jax.numpy as jnp
from jax import lax
from jax.experimental import pallas as pl
from jax.experimental.pallas import tpu as pltpu
from jax.experimental.pallas import tpu_sc as plsc

VOCAB = 1000000
EMBED = 64
B = 1024
L = 200
N = B * L

_INFO = plsc.get_sparse_core_info()
NC = _INFO.num_cores
NS = _INFO.num_subcores
NW = NC * NS
PER_W = N // NW
SUB = 128
CH = 640
NG = CH // SUB
NCH = PER_W // CH
VL = 16
NV = EMBED // VL

_DNUMS = lax.GatherDimensionNumbers(
    offset_dims=(), collapsed_slice_dims=(0,), start_index_map=(0,))


def _shuffle(v, idx):
    return lax.gather(v, idx[:, None], _DNUMS, slice_sizes=(1,),
                      mode=lax.GatherScatterMode.PROMISE_IN_BOUNDS)


def _rsqrt_nr(x):
    i = plsc.bitcast(x, jnp.int32)
    i = jnp.int32(0x5F3759DF) - lax.shift_right_arithmetic(i, jnp.int32(1))
    y = plsc.bitcast(i, jnp.float32)
    half = x * 0.5
    for _ in range(3):
        y = y * (1.5 - half * y * y)
    return y


def _ln_kernel(ids_hbm, table_hbm, gamma_hbm, beta_hbm, out_hbm,
               idx0, idx1, rows0, rows1, gb_v, sg0, sg1, so0, so1):
    wid = lax.axis_index("s") * NC + lax.axis_index("c")
    base = wid * PER_W
    idx = (idx0, idx1)
    rows = (rows0, rows1)
    sg = (sg0, sg1)
    so = (so0, so1)

    pltpu.sync_copy(gamma_hbm, gb_v.at[0])
    pltpu.sync_copy(beta_hbm, gb_v.at[1])
    gamma = [gb_v[0, pl.ds(k * VL, VL)] for k in range(NV)]
    beta = [gb_v[1, pl.ds(k * VL, VL)] for k in range(NV)]

    lanes = lax.iota(jnp.int32, VL)
    bfly = [lanes ^ s for s in (1, 2, 4, 8)]

    def gather_cp(b, g):
        return pltpu.make_async_copy(
            table_hbm.at[idx[b].at[pl.ds(g * SUB, SUB)]],
            rows[b].at[pl.ds(g * SUB, SUB)], sg[b])

    def fire(b, c):
        pltpu.sync_copy(ids_hbm.at[pl.ds(base + c * CH, CH)], idx[b])
        for g in range(NG):
            gather_cp(b, g).start()

    def wait_gather(b):
        for g in range(NG):
            gather_cp(b, g).wait()

    def out_cp(b, c):
        return pltpu.make_async_copy(
            rows[b], out_hbm.at[pl.ds(base + c * CH, CH)], so[b])

    def compute(b):
        rv = rows[b]

        def row(r, _):
            x = [rv[r, pl.ds(k * VL, VL)] for k in range(NV)]
            s = (x[0] + x[1]) + (x[2] + x[3])
            q = ((x[0] * x[0] + x[1] * x[1])
                 + (x[2] * x[2] + x[3] * x[3]))
            for p in bfly:
                s = s + _shuffle(s, p)
                q = q + _shuffle(q, p)
            mean = s * (1.0 / EMBED)
            var = q * (1.0 / EMBED) - mean * mean
            inv = _rsqrt_nr(var + 1e-12)
            for k in range(NV):
                a = inv * gamma[k]
                rv[r, pl.ds(k * VL, VL)] = x[k] * a + (beta[k] - mean * a)
            return 0

        lax.fori_loop(0, CH, row, 0, unroll=2)

    fire(0, 0)
    for c in range(NCH):
        b = c & 1
        if c + 1 < NCH:
            if c >= 1:
                out_cp(1 - b, c - 1).wait()
            fire(1 - b, c + 1)
        wait_gather(b)
        compute(b)
        out_cp(b, c).start()
    out_cp(NCH & 1, NCH - 2).wait()
    out_cp(1 - (NCH & 1), NCH - 1).wait()


@jax.jit
def _run(ids_flat, table, gamma, beta):
    mesh = plsc.VectorSubcoreMesh(core_axis_name="c", subcore_axis_name="s")
    out = pl.kernel(
        _ln_kernel,
        mesh=mesh,
        out_type=jax.ShapeDtypeStruct((N, EMBED), jnp.float32),
        scratch_types=[
            pltpu.VMEM((CH,), jnp.int32),
            pltpu.VMEM((CH,), jnp.int32),
            pltpu.VMEM((CH, EMBED), jnp.float32),
            pltpu.VMEM((CH, EMBED), jnp.float32),
            pltpu.VMEM((2, EMBED), jnp.float32),
            pltpu.SemaphoreType.DMA,
            pltpu.SemaphoreType.DMA,
            pltpu.SemaphoreType.DMA,
            pltpu.SemaphoreType.DMA,
        ],
        compiler_params=pltpu.CompilerParams(
            needs_layout_passes=False, use_tc_tiling_on_sc=False),
    )(ids_flat, table, gamma, beta)
    return out.reshape(B, L, EMBED)


def kernel(input_ids, word_table, ln_gamma, ln_beta):
    ids_flat = input_ids.astype(jnp.int32).reshape(N)
    return _run(ids_flat, word_table, ln_gamma, ln_beta)


# speedup vs baseline: 1.3077x; 1.0045x over previous
import functools

import jax
import jax.numpy as jnp
from jax import lax
from jax.experimental import pallas as pl
from jax.experimental.pallas import tpu as pltpu
from jax.experimental.pallas import tpu_sc as plsc

VOCAB = 1000000
EMBED = 64
B = 1024
L = 200
N = B * L

_INFO = plsc.get_sparse_core_info()
NC = _INFO.num_cores
NS = _INFO.num_subcores
NW = NC * NS
ROWS_W = B // NW
CHB = 4
NCH = ROWS_W // CHB
SUBG = 40
NGR = L // SUBG
VL = 16
NV = EMBED // VL

_DNUMS = lax.GatherDimensionNumbers(
    offset_dims=(), collapsed_slice_dims=(0,), start_index_map=(0,))


def _shuffle(v, idx):
    return lax.gather(v, idx[:, None], _DNUMS, slice_sizes=(1,),
                      mode=lax.GatherScatterMode.PROMISE_IN_BOUNDS)


def _rsqrt_nr(x):
    i = plsc.bitcast(x, jnp.int32)
    i = jnp.int32(0x5F3759DF) - lax.shift_right_arithmetic(i, jnp.int32(1))
    y = plsc.bitcast(i, jnp.float32)
    half = x * 0.5
    for _ in range(3):
        y = y * (1.5 - half * y * y)
    return y


def _ln_kernel(ids_hbm, table_hbm, gamma_hbm, beta_hbm, out_hbm,
               idx_v, rows0, rows1, gb_v, sg0, sg1, so0, so1):
    wid = lax.axis_index("s") * NC + lax.axis_index("c")
    base = wid * ROWS_W
    rows = (rows0, rows1)
    sg = (sg0, sg1)
    so = (so0, so1)

    pltpu.sync_copy(ids_hbm.at[pl.ds(base, ROWS_W)], idx_v)
    pltpu.sync_copy(gamma_hbm, gb_v.at[0])
    pltpu.sync_copy(beta_hbm, gb_v.at[1])
    gamma = [gb_v[0, pl.ds(k * VL, VL)] for k in range(NV)]
    beta = [gb_v[1, pl.ds(k * VL, VL)] for k in range(NV)]

    lanes = lax.iota(jnp.int32, VL)
    bfly = [lanes ^ s for s in (1, 2, 4, 8)]

    def gather_cp(b, c, i, g):
        return pltpu.make_async_copy(
            table_hbm.at[idx_v.at[c * CHB + i, pl.ds(g * SUBG, SUBG)]],
            rows[b].at[i, pl.ds(g * SUBG, SUBG)], sg[b])

    def fire(b, c):
        for i in range(CHB):
            for g in range(NGR):
                gather_cp(b, c, i, g).start()

    def wait_gather(b, c):
        for i in range(CHB):
            for g in range(NGR):
                gather_cp(b, c, i, g).wait()

    def out_cp(b, c):
        return pltpu.make_async_copy(
            rows[b], out_hbm.at[pl.ds(base + c * CHB, CHB)], so[b])

    def compute(b):
        rv = rows[b]

        def make_row(i):
            def row(l, _):
                x = [rv[i, l, pl.ds(k * VL, VL)] for k in range(NV)]
                s = (x[0] + x[1]) + (x[2] + x[3])
                q = ((x[0] * x[0] + x[1] * x[1])
                     + (x[2] * x[2] + x[3] * x[3]))
                for p in bfly:
                    s = s + _shuffle(s, p)
                    q = q + _shuffle(q, p)
                mean = s * (1.0 / EMBED)
                var = q * (1.0 / EMBED) - mean * mean
                inv = _rsqrt_nr(var + 1e-12)
                for k in range(NV):
                    a = inv * gamma[k]
                    rv[i, l, pl.ds(k * VL, VL)] = (
                        x[k] * a + (beta[k] - mean * a))
                return 0
            return row

        for i in range(CHB):
            lax.fori_loop(0, L, make_row(i), 0, unroll=2)

    fire(0, 0)
    for c in range(NCH):
        b = c & 1
        if c + 1 < NCH:
            if c >= 1:
                out_cp(1 - b, c - 1).wait()
            fire(1 - b, c + 1)
        wait_gather(b, c)
        compute(b)
        out_cp(b, c).start()
    out_cp(NCH & 1, NCH - 2).wait()
    out_cp(1 - (NCH & 1), NCH - 1).wait()


@jax.jit
def _run(ids2d, table, gamma, beta):
    mesh = plsc.VectorSubcoreMesh(core_axis_name="c", subcore_axis_name="s")
    return pl.kernel(
        _ln_kernel,
        mesh=mesh,
        out_type=jax.ShapeDtypeStruct((B, L, EMBED), jnp.float32),
        scratch_types=[
            pltpu.VMEM((ROWS_W, L), jnp.int32),
            pltpu.VMEM((CHB, L, EMBED), jnp.float32),
            pltpu.VMEM((CHB, L, EMBED), jnp.float32),
            pltpu.VMEM((2, EMBED), jnp.float32),
            pltpu.SemaphoreType.DMA,
            pltpu.SemaphoreType.DMA,
            pltpu.SemaphoreType.DMA,
            pltpu.SemaphoreType.DMA,
        ],
        compiler_params=pltpu.CompilerParams(
            needs_layout_passes=False, use_tc_tiling_on_sc=False,
            skip_device_barrier=True),
    )(ids2d, table, gamma, beta)


def kernel(input_ids, word_table, ln_gamma, ln_beta):
    return _run(input_ids.astype(jnp.int32), word_table, ln_gamma, ln_beta)


# speedup vs baseline: 1.3251x; 1.0133x over previous
import functools

import jax
import jax.numpy as jnp
from jax import lax
from jax.experimental import pallas as pl
from jax.experimental.pallas import tpu as pltpu
from jax.experimental.pallas import tpu_sc as plsc

VOCAB = 1000000
EMBED = 64
B = 1024
L = 200
N = B * L

_INFO = plsc.get_sparse_core_info()
NC = _INFO.num_cores
NS = _INFO.num_subcores
NW = NC * NS
PER_W = N // NW
SUB = 128
CH = 640
NG = CH // SUB
NCH = PER_W // CH
IROWS = PER_W // SUB

TC_GRID = 32
RPB = B // TC_GRID
EPB = RPB * L


def _gather_kernel(ids_hbm, table_hbm, out_hbm, idx_v, rows0, rows1,
                   sg0, sg1, so0, so1):
    wid = lax.axis_index("s") * NC + lax.axis_index("c")
    base = wid * PER_W
    rows = (rows0, rows1)
    sg = (sg0, sg1)
    so = (so0, so1)

    pltpu.sync_copy(ids_hbm.at[pl.ds(wid * IROWS, IROWS)], idx_v)

    def gather_cp(b, c, g):
        return pltpu.make_async_copy(
            table_hbm.at[idx_v.at[c * NG + g]],
            rows[b].at[pl.ds(g * SUB, SUB)], sg[b])

    def fire(b, c):
        for g in range(NG):
            gather_cp(b, c, g).start()

    def wait_gather(b, c):
        for g in range(NG):
            gather_cp(b, c, g).wait()

    def out_cp(b, c):
        return pltpu.make_async_copy(
            rows[b], out_hbm.at[pl.ds(base + c * CH, CH)], so[b])

    fire(0, 0)
    for c in range(NCH):
        b = c & 1
        if c + 1 < NCH:
            if c >= 1:
                out_cp(1 - b, c - 1).wait()
            fire(1 - b, c + 1)
        wait_gather(b, c)
        out_cp(b, c).start()
    out_cp(NCH & 1, NCH - 2).wait()
    out_cp(1 - (NCH & 1), NCH - 1).wait()


def _ln_kernel(g_ref, gamma_ref, beta_ref, out_ref):
    x = g_ref[...]
    mean = jnp.mean(x, axis=-1, keepdims=True)
    d = x - mean
    var = jnp.mean(d * d, axis=-1, keepdims=True)
    normed = d * lax.rsqrt(var + 1e-12)
    y = normed * gamma_ref[0] + beta_ref[0]
    out_ref[...] = y.reshape(RPB, L, EMBED)


@jax.jit
def _run(ids2d, table, gamma, beta):
    ids128 = jnp.minimum(ids2d, VOCAB - 1).reshape(N // SUB, SUB)
    mesh = plsc.VectorSubcoreMesh(core_axis_name="c", subcore_axis_name="s")
    gathered = pl.kernel(
        _gather_kernel,
        mesh=mesh,
        out_type=jax.ShapeDtypeStruct((N, EMBED), jnp.float32),
        scratch_types=[
            pltpu.VMEM((IROWS, SUB), jnp.int32),
            pltpu.VMEM((CH, EMBED), jnp.float32),
            pltpu.VMEM((CH, EMBED), jnp.float32),
            pltpu.SemaphoreType.DMA,
            pltpu.SemaphoreType.DMA,
            pltpu.SemaphoreType.DMA,
            pltpu.SemaphoreType.DMA,
        ],
        compiler_params=pltpu.CompilerParams(
            needs_layout_passes=False, use_tc_tiling_on_sc=False,
            skip_device_barrier=True),
    )(ids128, table)

    out = pl.pallas_call(
        _ln_kernel,
        grid=(TC_GRID,),
        in_specs=[
            pl.BlockSpec((EPB, EMBED), lambda b: (b, 0)),
            pl.BlockSpec((1, EMBED), lambda b: (0, 0)),
            pl.BlockSpec((1, EMBED), lambda b: (0, 0)),
        ],
        out_specs=pl.BlockSpec((RPB, L, EMBED), lambda b: (b, 0, 0)),
        out_shape=jax.ShapeDtypeStruct((B, L, EMBED), jnp.float32),
        compiler_params=pltpu.CompilerParams(
            dimension_semantics=("arbitrary",)),
    )(gathered, gamma.reshape(1, EMBED), beta.reshape(1, EMBED))
    return out


def kernel(input_ids, word_table, ln_gamma, ln_beta):
    return _run(input_ids.astype(jnp.int32), word_table, ln_gamma, ln_beta)


# speedup vs baseline: 1.4506x; 1.0947x over previous
import functools

import jax
import jax.numpy as jnp
from jax import lax
from jax.experimental import pallas as pl
from jax.experimental.pallas import tpu as pltpu
from jax.experimental.pallas import tpu_sc as plsc

VOCAB = 1000000
EMBED = 64
B = 1024
L = 200
N = B * L

_INFO = plsc.get_sparse_core_info()
NC = _INFO.num_cores
NS = _INFO.num_subcores
NW = NC * NS
PER_W = N // NW
SUB = 128
CH = 640
NG = CH // SUB
NCH = PER_W // CH
IROWS = PER_W // SUB

TC_GRID = 32
RPB = B // TC_GRID
EPB = RPB * L


def _gather_kernel(ids_hbm, table_hbm, out_hbm, idx_v, rows0, rows1,
                   sg0, sg1, so0, so1):
    wid = lax.axis_index("s") * NC + lax.axis_index("c")
    base = wid * PER_W
    rows = (rows0, rows1)
    sg = (sg0, sg1)
    so = (so0, so1)

    pltpu.sync_copy(ids_hbm.at[pl.ds(wid * IROWS, IROWS)], idx_v)

    def gather_cp(b, c, g):
        return pltpu.make_async_copy(
            table_hbm.at[idx_v.at[c * NG + g]],
            rows[b].at[pl.ds(g * SUB, SUB)], sg[b])

    def fire(b, c):
        for g in range(NG):
            gather_cp(b, c, g).start()

    def wait_gather(b, c):
        for g in range(NG):
            gather_cp(b, c, g).wait()

    def out_cp(b, c):
        return pltpu.make_async_copy(
            rows[b],
            out_hbm.at[pl.ds(base + c * CH, CH), pl.ds(0, EMBED)], so[b])

    fire(0, 0)
    for c in range(NCH):
        b = c & 1
        if c + 1 < NCH:
            if c >= 1:
                out_cp(1 - b, c - 1).wait()
            fire(1 - b, c + 1)
        wait_gather(b, c)
        out_cp(b, c).start()
    out_cp(NCH & 1, NCH - 2).wait()
    out_cp(1 - (NCH & 1), NCH - 1).wait()


def _ln_kernel(g_ref, gamma_ref, beta_ref, out_ref):
    x = g_ref[...][:, :EMBED]
    mean = jnp.mean(x, axis=-1, keepdims=True)
    d = x - mean
    var = jnp.mean(d * d, axis=-1, keepdims=True)
    normed = d * lax.rsqrt(var + 1e-12)
    y = normed * gamma_ref[0] + beta_ref[0]
    out_ref[...] = y.reshape(RPB, L, EMBED)


@jax.jit
def _run(ids2d, table, gamma, beta):
    ids128 = jnp.minimum(ids2d, VOCAB - 1).reshape(N // SUB, SUB)
    mesh = plsc.VectorSubcoreMesh(core_axis_name="c", subcore_axis_name="s")
    gathered = pl.kernel(
        _gather_kernel,
        mesh=mesh,
        out_type=jax.ShapeDtypeStruct((N, 2 * EMBED), jnp.float32),
        scratch_types=[
            pltpu.VMEM((IROWS, SUB), jnp.int32),
            pltpu.VMEM((CH, EMBED), jnp.float32),
            pltpu.VMEM((CH, EMBED), jnp.float32),
            pltpu.SemaphoreType.DMA,
            pltpu.SemaphoreType.DMA,
            pltpu.SemaphoreType.DMA,
            pltpu.SemaphoreType.DMA,
        ],
        compiler_params=pltpu.CompilerParams(
            needs_layout_passes=False, use_tc_tiling_on_sc=False,
            skip_device_barrier=True),
    )(ids128, table)

    out = pl.pallas_call(
        _ln_kernel,
        grid=(TC_GRID,),
        in_specs=[
            pl.BlockSpec((EPB, 2 * EMBED), lambda b: (b, 0)),
            pl.BlockSpec((1, EMBED), lambda b: (0, 0)),
            pl.BlockSpec((1, EMBED), lambda b: (0, 0)),
        ],
        out_specs=pl.BlockSpec((RPB, L, EMBED), lambda b: (b, 0, 0)),
        out_shape=jax.ShapeDtypeStruct((B, L, EMBED), jnp.float32),
        compiler_params=pltpu.CompilerParams(
            dimension_semantics=("arbitrary",)),
    )(gathered, gamma.reshape(1, EMBED), beta.reshape(1, EMBED))
    return out


def kernel(input_ids, word_table, ln_gamma, ln_beta):
    return _run(input_ids.astype(jnp.int32), word_table, ln_gamma, ln_beta)


# speedup vs baseline: 2.1375x; 1.4736x over previous
import functools

import jax
import jax.numpy as jnp
from jax import lax
from jax.experimental import layout as jex_layout
from jax.experimental import pallas as pl
from jax.experimental.pallas import tpu as pltpu
from jax.experimental.pallas import tpu_sc as plsc

VOCAB = 1000000
EMBED = 64
B = 1024
L = 200
N = B * L

_INFO = plsc.get_sparse_core_info()
NC = _INFO.num_cores
NS = _INFO.num_subcores
NW = NC * NS
PER_W = N // NW
SUB = 128
CH = 640
NG = CH // SUB
NCH = PER_W // CH
IROWS = PER_W // SUB

TC_GRID = 32
RPB = B // TC_GRID
EPB = RPB * L


def _gather_kernel(ids_hbm, table_hbm, out_hbm, idx_v, rows0, rows1,
                   sg0, sg1, so0, so1):
    wid = lax.axis_index("s") * NC + lax.axis_index("c")
    base = wid * PER_W
    rows = (rows0, rows1)
    sg = (sg0, sg1)
    so = (so0, so1)

    pltpu.sync_copy(ids_hbm.at[pl.ds(wid * IROWS, IROWS)], idx_v)

    def gather_cp(b, c, g):
        return pltpu.make_async_copy(
            table_hbm.at[idx_v.at[c * NG + g]],
            rows[b].at[pl.ds(g * SUB, SUB)], sg[b])

    def fire(b, c):
        for g in range(NG):
            gather_cp(b, c, g).start()

    def wait_gather(b, c):
        for g in range(NG):
            gather_cp(b, c, g).wait()

    def out_cp(b, c):
        return pltpu.make_async_copy(
            rows[b],
            out_hbm.at[pl.ds(base + c * CH, CH), pl.ds(0, EMBED)], so[b])

    fire(0, 0)
    for c in range(NCH):
        b = c & 1
        if c + 1 < NCH:
            if c >= 1:
                out_cp(1 - b, c - 1).wait()
            fire(1 - b, c + 1)
        wait_gather(b, c)
        out_cp(b, c).start()
    out_cp(NCH & 1, NCH - 2).wait()
    out_cp(1 - (NCH & 1), NCH - 1).wait()


def _ln_kernel(g_ref, gamma_ref, beta_ref, out_ref):
    x = g_ref[...][:, :EMBED]
    mean = jnp.mean(x, axis=-1, keepdims=True)
    d = x - mean
    var = jnp.mean(d * d, axis=-1, keepdims=True)
    normed = d * lax.rsqrt(var + 1e-12)
    y = normed * gamma_ref[0] + beta_ref[0]
    out_ref[...] = y.reshape(RPB, L, EMBED)


@jax.jit
def _run(ids2d, table, gamma, beta):
    ids128 = jnp.minimum(ids2d, VOCAB - 1).reshape(N // SUB, SUB)
    table = jex_layout.with_layout_constraint(
        table,
        jex_layout.Layout(major_to_minor=(0, 1), tiling=((8,),)),
    )
    mesh = plsc.VectorSubcoreMesh(core_axis_name="c", subcore_axis_name="s")
    gathered = pl.kernel(
        _gather_kernel,
        mesh=mesh,
        out_type=jax.ShapeDtypeStruct((N, 2 * EMBED), jnp.float32),
        scratch_types=[
            pltpu.VMEM((IROWS, SUB), jnp.int32),
            pltpu.VMEM((CH, EMBED), jnp.float32),
            pltpu.VMEM((CH, EMBED), jnp.float32),
            pltpu.SemaphoreType.DMA,
            pltpu.SemaphoreType.DMA,
            pltpu.SemaphoreType.DMA,
            pltpu.SemaphoreType.DMA,
        ],
        compiler_params=pltpu.CompilerParams(
            needs_layout_passes=False, use_tc_tiling_on_sc=False,
            skip_device_barrier=True),
    )(ids128, table)

    out = pl.pallas_call(
        _ln_kernel,
        grid=(TC_GRID,),
        in_specs=[
            pl.BlockSpec((EPB, 2 * EMBED), lambda b: (b, 0)),
            pl.BlockSpec((1, EMBED), lambda b: (0, 0)),
            pl.BlockSpec((1, EMBED), lambda b: (0, 0)),
        ],
        out_specs=pl.BlockSpec((RPB, L, EMBED), lambda b: (b, 0, 0)),
        out_shape=jax.ShapeDtypeStruct((B, L, EMBED), jnp.float32),
        compiler_params=pltpu.CompilerParams(
            dimension_semantics=("arbitrary",)),
    )(gathered, gamma.reshape(1, EMBED), beta.reshape(1, EMBED))
    return out


def kernel(input_ids, word_table, ln_gamma, ln_beta):
    return _run(input_ids.astype(jnp.int32), word_table, ln_gamma, ln_beta)
